# baseline (speedup 1.0000x reference)
import dataclasses
import functools

import numpy as np

import jax
import jax.numpy as jnp
from jax import lax
from jax.experimental import pallas as pl
from jax.experimental.pallas import tpu as pltpu
from jax.experimental.pallas import tpu_sc as plsc

N = 10000
D = 128
N_PAD = 10240
NC, NS = 2, 16
NW = NC * NS
CHUNK = 64
NBUF = 3
_LEAD = NBUF // 2
_LAG = NBUF - _LEAD
SEG = 48
CAP = (240, 96)
CAPMAX = max(CAP)

_MESH = plsc.VectorSubcoreMesh(
    core_axis_name="c", subcore_axis_name="s", num_cores=NC, num_subcores=NS
)

_SC_CP = pltpu.CompilerParams()
if "needs_layout_passes" in pltpu.CompilerParams.__dataclass_fields__:
    _SC_CP = dataclasses.replace(_SC_CP, needs_layout_passes=False)

_TOTAL_CHUNKS = NS * (CAP[0] + CAP[1])


def _chunk_map():
    m = np.zeros((NW, CAPMAX), np.int32)
    k = 0
    for wid in range(NW):
        cap = CAP[wid % NC]
        m[wid, :cap] = np.arange(k, k + cap)
        k += cap
    assert k == _TOTAL_CHUNKS
    return m


def _sc_count(dst_p):
    e_pad = dst_p.shape[0]
    e_pt = e_pad // NW

    @functools.partial(
        pl.kernel,
        out_type=jax.ShapeDtypeStruct((NW, N_PAD), jnp.float32),
        mesh=_MESH,
        scratch_types=[
            pltpu.VMEM((e_pt,), jnp.int32),
            pltpu.VMEM((N_PAD,), jnp.float32),
        ],
        compiler_params=_SC_CP,
    )
    def k(dst_hbm, out_hbm, dst_v, cnt_v):
        c = lax.axis_index("c")
        s = lax.axis_index("s")
        wid = s * NC + c
        zero = jnp.zeros((16,), jnp.float32)

        @pl.loop(0, N_PAD // 16)
        def _(i):
            cnt_v[pl.ds(i * 16, 16)] = zero

        pltpu.sync_copy(dst_hbm.at[pl.ds(wid * e_pt, e_pt)], dst_v)
        ones = jnp.ones((16,), jnp.float32)

        @pl.loop(0, e_pt // 16)
        def _(i):
            idx = dst_v[pl.ds(i * 16, 16)]
            plsc.addupdate_scatter(cnt_v, [idx], ones)

        pltpu.sync_copy(cnt_v, out_hbm.at[wid])

    return k(dst_p)


def _sc_aggregate(g, src_3d, dst_3d, zeros):
    stripe = N_PAD // NS

    @functools.partial(
        pl.kernel,
        out_type=jax.ShapeDtypeStruct((NC * N_PAD, D), jnp.float32),
        mesh=_MESH,
        scratch_types=[
            pltpu.VMEM((2, SEG, CHUNK), jnp.int32),
            pltpu.VMEM((2, SEG, CHUNK), jnp.int32),
            pltpu.VMEM((NBUF, CHUNK, D), jnp.float32),
            pltpu.VMEM_SHARED((N_PAD, D), jnp.float32),
            pltpu.SemaphoreType.DMA((NBUF,)),
            pltpu.SemaphoreType.DMA((NBUF,)),
            pltpu.SemaphoreType.DMA((2,)),
        ],
    )
    def k(g_hbm, src_hbm, dst_hbm, z_hbm, out_hbm,
          src_v, dst_v, rows_v, acc, gat, scat, isem):
        c = lax.axis_index("c")
        s = lax.axis_index("s")
        wid = s * NC + c
        r0 = s * stripe

        def issue_idx(seg, p):
            pltpu.async_copy(
                src_hbm.at[wid, pl.ds(seg * SEG, SEG)], src_v.at[p], isem.at[p])
            pltpu.async_copy(
                dst_hbm.at[wid, pl.ds(seg * SEG, SEG)], dst_v.at[p], isem.at[p])

        def wait_idx(seg, p):
            pltpu.make_async_copy(
                src_hbm.at[wid, pl.ds(seg * SEG, SEG)], src_v.at[p],
                isem.at[p]).wait()
            pltpu.make_async_copy(
                dst_hbm.at[wid, pl.ds(seg * SEG, SEG)], dst_v.at[p],
                isem.at[p]).wait()

        issue_idx(0, 0)
        pltpu.sync_copy(z_hbm.at[pl.ds(r0, stripe)], acc.at[pl.ds(r0, stripe)])
        wait_idx(0, 0)
        plsc.subcore_barrier()

        def issue_gather(p, t, b):
            pltpu.async_copy(g_hbm.at[src_v.at[p, t]], rows_v.at[b], gat.at[b])

        def wait_gather(p, t, b):
            pltpu.make_async_copy(
                g_hbm.at[src_v.at[p, t]], rows_v.at[b], gat.at[b]).wait()

        def issue_scatter(p, t, b):
            pltpu.async_copy(rows_v.at[b], acc.at[dst_v.at[p, t]],
                             scat.at[b], add=True)

        def wait_scatter(p, t, b):
            pltpu.make_async_copy(
                rows_v.at[b], acc.at[dst_v.at[p, t]], scat.at[b]).wait()

        def run(cap):
            nseg = cap // SEG
            for seg in range(nseg):
                p = seg % 2
                if seg + 1 < nseg:
                    issue_idx(seg + 1, 1 - p)
                if seg > 0:
                    wait_idx(seg, p)

                for t in range(_LEAD):
                    issue_gather(p, t, t)
                for t in range(_LAG):
                    wait_gather(p, t, t % NBUF)
                    issue_scatter(p, t, t % NBUF)
                    issue_gather(p, t + _LEAD, (t + _LEAD) % NBUF)

                @pl.loop(0, (SEG - NBUF) // NBUF)
                def _(i):
                    for kk in range(NBUF):
                        t = _LAG + i * NBUF + kk
                        b = (_LAG + kk) % NBUF
                        bu = kk
                        wait_gather(p, t, b)
                        issue_scatter(p, t, b)
                        wait_scatter(p, t - _LAG, bu)
                        issue_gather(p, t + _LEAD, bu)

                for tt in range(_LEAD):
                    t = SEG - _LEAD + tt
                    wait_gather(p, t, t % NBUF)
                    issue_scatter(p, t, t % NBUF)
                for b in range(NBUF):
                    wait_scatter(p, 0, b)

        @pl.when(c == 0)
        def _():
            run(CAP[0])

        @pl.when(c == 1)
        def _():
            run(CAP[1])

        plsc.subcore_barrier()
        pltpu.sync_copy(
            acc.at[pl.ds(r0, stripe)], out_hbm.at[pl.ds(c * N_PAD + r0, stripe)]
        )

    return k(g, src_3d, dst_3d, zeros)


def _dinv_from_counts(cnt):
    deg = jnp.sum(cnt, axis=0) + 1.0
    return lax.rsqrt(deg)


def _tc_g1(cnt, x_p, W1):
    def body(cnt_ref, x_ref, w_ref, o_ref):
        dinv = _dinv_from_counts(cnt_ref[...])
        h = jnp.dot(
            x_ref[...], w_ref[...],
            preferred_element_type=jnp.float32,
            precision=lax.Precision.HIGHEST,
        )
        o_ref[...] = h * dinv[:, None]

    return pl.pallas_call(
        body, out_shape=jax.ShapeDtypeStruct((N_PAD, D), jnp.float32)
    )(cnt, x_p, W1)


def _tc_mid(cnt, agg, g1, b1, W2):
    def body(cnt_ref, agg_ref, g1_ref, b1_ref, w2_ref, o_ref):
        dinv = _dinv_from_counts(cnt_ref[...])
        ssum = agg_ref[:N_PAD, :] + agg_ref[N_PAD:, :] + g1_ref[...]
        h1 = jnp.maximum(ssum * dinv[:, None] + b1_ref[...][None, :], 0.0)
        h2 = jnp.dot(
            h1, w2_ref[...],
            preferred_element_type=jnp.float32,
            precision=lax.Precision.HIGHEST,
        )
        o_ref[...] = h2 * dinv[:, None]

    return pl.pallas_call(
        body, out_shape=jax.ShapeDtypeStruct((N_PAD, D), jnp.float32)
    )(cnt, agg, g1, b1, W2)


def _tc_final(cnt, agg, g2, b2):
    def body(cnt_ref, agg_ref, g2_ref, b2_ref, o_ref):
        dinv = _dinv_from_counts(cnt_ref[...])
        ssum = agg_ref[:N_PAD, :] + agg_ref[N_PAD:, :] + g2_ref[...]
        o_ref[...] = ssum * dinv[:, None] + b2_ref[...][None, :]

    return pl.pallas_call(
        body, out_shape=jax.ShapeDtypeStruct((N_PAD, D), jnp.float32)
    )(cnt, agg, g2, b2)


def kernel(x, edge_index, W1, b1, W2, b2):
    e = edge_index.shape[1]
    e_pad = _TOTAL_CHUNKS * CHUNK
    assert e_pad >= e
    pad = jnp.full((e_pad - e,), N, dtype=jnp.int32)
    src_p = jnp.concatenate([edge_index[0], pad])
    dst_p = jnp.concatenate([edge_index[1], pad])
    cmap = jnp.asarray(_chunk_map())
    src_3d = jnp.take(src_p.reshape(_TOTAL_CHUNKS, CHUNK), cmap, axis=0)
    dst_3d = jnp.take(dst_p.reshape(_TOTAL_CHUNKS, CHUNK), cmap, axis=0)
    x_p = jnp.pad(x, ((0, N_PAD - N), (0, 0)))
    zeros = jnp.zeros((N_PAD, D), jnp.float32)

    cnt = _sc_count(dst_p)
    g1 = _tc_g1(cnt, x_p, W1)
    agg1 = _sc_aggregate(g1, src_3d, dst_3d, zeros)
    g2 = _tc_mid(cnt, agg1, g1, b1, W2)
    agg2 = _sc_aggregate(g2, src_3d, dst_3d, zeros)
    out = _tc_final(cnt, agg2, g2, b2)
    return out[:N]


# speedup vs baseline: 2.7963x; 2.7963x over previous
import dataclasses
import functools

import jax
import jax.numpy as jnp
from jax import lax
from jax.experimental import pallas as pl
from jax.experimental.pallas import tpu as pltpu
from jax.experimental.pallas import tpu_sc as plsc

N = 10000
D = 128
N_PAD = 10240
NC, NS = 2, 16
NW = NC * NS
CHUNK = 64
NBUF = 4
_LEAD = NBUF // 2
_LAG = NBUF - _LEAD
SEG = 16

_MESH = plsc.VectorSubcoreMesh(
    core_axis_name="c", subcore_axis_name="s", num_cores=NC, num_subcores=NS
)

_SC_CP = pltpu.CompilerParams()
if "needs_layout_passes" in pltpu.CompilerParams.__dataclass_fields__:
    _SC_CP = dataclasses.replace(_SC_CP, needs_layout_passes=False)


def _pad_edges(e):
    per = NW * CHUNK
    chunks_pt = (e + per - 1) // per
    chunks_pt = max(((chunks_pt + SEG - 1) // SEG) * SEG, SEG)
    return chunks_pt * per


def _sc_count(dst_p):
    e_pad = dst_p.shape[0]
    e_pt = e_pad // NW

    @functools.partial(
        pl.kernel,
        out_type=jax.ShapeDtypeStruct((NW, N_PAD), jnp.float32),
        mesh=_MESH,
        scratch_types=[
            pltpu.VMEM((e_pt,), jnp.int32),
            pltpu.VMEM((N_PAD,), jnp.float32),
        ],
        compiler_params=_SC_CP,
    )
    def k(dst_hbm, out_hbm, dst_v, cnt_v):
        c = lax.axis_index("c")
        s = lax.axis_index("s")
        wid = s * NC + c
        zero = jnp.zeros((16,), jnp.float32)

        @pl.loop(0, N_PAD // 16)
        def _(i):
            cnt_v[pl.ds(i * 16, 16)] = zero

        pltpu.sync_copy(dst_hbm.at[pl.ds(wid * e_pt, e_pt)], dst_v)
        ones = jnp.ones((16,), jnp.float32)

        @pl.loop(0, e_pt // 16)
        def _(i):
            idx = dst_v[pl.ds(i * 16, 16)]
            plsc.addupdate_scatter(cnt_v, [idx], ones)

        pltpu.sync_copy(cnt_v, out_hbm.at[wid])

    return k(dst_p)


def _sc_aggregate(g, idx_2d, zeros):
    chunks = idx_2d.shape[1]
    stripe = N_PAD // NS
    nseg = chunks // SEG
    assert chunks % SEG == 0 and SEG % NBUF == 0 and SEG >= 2 * NBUF

    @functools.partial(
        pl.kernel,
        out_type=jax.ShapeDtypeStruct((NC * N_PAD, D), jnp.float32),
        mesh=_MESH,
        scratch_types=[
            pltpu.VMEM((2, SEG, 2 * CHUNK), jnp.int32),
            pltpu.VMEM((NBUF, CHUNK, D), jnp.float32),
            pltpu.VMEM_SHARED((N_PAD, D), jnp.float32),
            pltpu.SemaphoreType.DMA((NBUF,)),
            pltpu.SemaphoreType.DMA((NBUF,)),
            pltpu.SemaphoreType.DMA((2,)),
        ],
    )
    def k(g_hbm, idx_hbm, z_hbm, out_hbm, idx_v, rows_v, acc, gat, scat, isem):
        c = lax.axis_index("c")
        s = lax.axis_index("s")
        wid = s * NC + c
        r0 = s * stripe

        def issue_idx(seg, p):
            pltpu.async_copy(idx_hbm.at[wid, pl.ds(seg * SEG, SEG)],
                             idx_v.at[p], isem.at[p])

        def wait_idx(seg, p):
            pltpu.make_async_copy(idx_hbm.at[wid, pl.ds(seg * SEG, SEG)],
                                  idx_v.at[p], isem.at[p]).wait()

        issue_idx(0, 0)
        pltpu.sync_copy(z_hbm.at[pl.ds(r0, stripe)], acc.at[pl.ds(r0, stripe)])
        wait_idx(0, 0)
        plsc.subcore_barrier()

        def issue_gather(p, t, b):
            pltpu.async_copy(
                g_hbm.at[idx_v.at[p, t, pl.ds(0, CHUNK)]], rows_v.at[b],
                gat.at[b])

        def wait_gather(p, t, b):
            pltpu.make_async_copy(
                g_hbm.at[idx_v.at[p, t, pl.ds(0, CHUNK)]], rows_v.at[b],
                gat.at[b]).wait()

        def issue_scatter(p, t, b):
            pltpu.async_copy(rows_v.at[b],
                             acc.at[idx_v.at[p, t, pl.ds(CHUNK, CHUNK)]],
                             scat.at[b], add=True)

        def wait_scatter(p, t, b):
            pltpu.make_async_copy(
                rows_v.at[b], acc.at[idx_v.at[p, t, pl.ds(CHUNK, CHUNK)]],
                scat.at[b]).wait()

        for seg in range(nseg):
            p = seg % 2
            if seg + 1 < nseg:
                issue_idx(seg + 1, 1 - p)
            if seg > 0:
                wait_idx(seg, p)

            for t in range(_LEAD):
                issue_gather(p, t, t)
            for t in range(_LAG):
                wait_gather(p, t, t % NBUF)
                issue_scatter(p, t, t % NBUF)
                issue_gather(p, t + _LEAD, (t + _LEAD) % NBUF)

            @pl.loop(0, (SEG - NBUF) // NBUF)
            def _(i):
                for kk in range(NBUF):
                    t = _LAG + i * NBUF + kk
                    b = (_LAG + kk) % NBUF
                    bu = kk
                    wait_gather(p, t, b)
                    issue_scatter(p, t, b)
                    wait_scatter(p, t - _LAG, bu)
                    issue_gather(p, t + _LEAD, bu)

            for tt in range(_LEAD):
                t = SEG - _LEAD + tt
                wait_gather(p, t, t % NBUF)
                issue_scatter(p, t, t % NBUF)
            for b in range(NBUF):
                wait_scatter(p, 0, b)

        plsc.subcore_barrier()
        pltpu.sync_copy(
            acc.at[pl.ds(r0, stripe)], out_hbm.at[pl.ds(c * N_PAD + r0, stripe)]
        )

    return k(g, idx_2d, zeros)


def _dinv_from_counts(cnt):
    deg = jnp.sum(cnt, axis=0) + 1.0
    return lax.rsqrt(deg)


def _tc_g1(cnt, x_p, W1):
    def body(cnt_ref, x_ref, w_ref, o_ref):
        dinv = _dinv_from_counts(cnt_ref[...])
        h = jnp.dot(
            x_ref[...], w_ref[...],
            preferred_element_type=jnp.float32,
            precision=lax.Precision.HIGHEST,
        )
        o_ref[...] = h * dinv[:, None]

    return pl.pallas_call(
        body, out_shape=jax.ShapeDtypeStruct((N_PAD, D), jnp.float32)
    )(cnt, x_p, W1)


def _tc_mid(cnt, agg, g1, b1, W2):
    def body(cnt_ref, agg_ref, g1_ref, b1_ref, w2_ref, o_ref):
        dinv = _dinv_from_counts(cnt_ref[...])
        ssum = agg_ref[:N_PAD, :] + agg_ref[N_PAD:, :] + g1_ref[...]
        h1 = jnp.maximum(ssum * dinv[:, None] + b1_ref[...][None, :], 0.0)
        h2 = jnp.dot(
            h1, w2_ref[...],
            preferred_element_type=jnp.float32,
            precision=lax.Precision.HIGHEST,
        )
        o_ref[...] = h2 * dinv[:, None]

    return pl.pallas_call(
        body, out_shape=jax.ShapeDtypeStruct((N_PAD, D), jnp.float32)
    )(cnt, agg, g1, b1, W2)


def _tc_final(cnt, agg, g2, b2):
    def body(cnt_ref, agg_ref, g2_ref, b2_ref, o_ref):
        dinv = _dinv_from_counts(cnt_ref[...])
        ssum = agg_ref[:N_PAD, :] + agg_ref[N_PAD:, :] + g2_ref[...]
        o_ref[...] = ssum * dinv[:, None] + b2_ref[...][None, :]

    return pl.pallas_call(
        body, out_shape=jax.ShapeDtypeStruct((N_PAD, D), jnp.float32)
    )(cnt, agg, g2, b2)


def kernel(x, edge_index, W1, b1, W2, b2):
    e = edge_index.shape[1]
    e_pad = _pad_edges(e)
    chunks = e_pad // (NW * CHUNK)
    pad = jnp.full((e_pad - e,), N, dtype=jnp.int32)
    src_p = jnp.concatenate([edge_index[0], pad])
    dst_p = jnp.concatenate([edge_index[1], pad])
    idx_2d = jnp.concatenate(
        [src_p.reshape(NW, chunks, CHUNK), dst_p.reshape(NW, chunks, CHUNK)],
        axis=2,
    )
    x_p = jnp.pad(x, ((0, N_PAD - N), (0, 0)))
    zeros = jnp.zeros((N_PAD, D), jnp.float32)

    cnt = _sc_count(dst_p)
    g1 = _tc_g1(cnt, x_p, W1)
    agg1 = _sc_aggregate(g1, idx_2d, zeros)
    g2 = _tc_mid(cnt, agg1, g1, b1, W2)
    agg2 = _sc_aggregate(g2, idx_2d, zeros)
    out = _tc_final(cnt, agg2, g2, b2)
    return out[:N]


# speedup vs baseline: 3.0917x; 1.1056x over previous
import dataclasses
import functools

import jax
import jax.numpy as jnp
from jax import lax
from jax.experimental import pallas as pl
from jax.experimental.pallas import tpu as pltpu
from jax.experimental.pallas import tpu_sc as plsc

N = 10000
D = 128
N_PAD = 10240
NC, NS = 2, 16
NW = NC * NS
CHUNK = 64
NBUF = 3
_LEAD = NBUF // 2
_LAG = NBUF - _LEAD

_MESH = plsc.VectorSubcoreMesh(
    core_axis_name="c", subcore_axis_name="s", num_cores=NC, num_subcores=NS
)

_SC_CP = pltpu.CompilerParams()
if "needs_layout_passes" in pltpu.CompilerParams.__dataclass_fields__:
    _SC_CP = dataclasses.replace(_SC_CP, needs_layout_passes=False)


def _pad_edges(e):
    per = NW * CHUNK
    chunks_pt = (e + per - 1) // per
    chunks_pt = max(((chunks_pt + NBUF - 1) // NBUF) * NBUF, 2 * NBUF)
    return chunks_pt * per


def _sc_count(dst_p):
    e_pad = dst_p.shape[0]
    e_pt = e_pad // NW

    @functools.partial(
        pl.kernel,
        out_type=jax.ShapeDtypeStruct((NW, N_PAD), jnp.float32),
        mesh=_MESH,
        scratch_types=[
            pltpu.VMEM((e_pt,), jnp.int32),
            pltpu.VMEM((N_PAD,), jnp.float32),
        ],
        compiler_params=_SC_CP,
    )
    def k(dst_hbm, out_hbm, dst_v, cnt_v):
        c = lax.axis_index("c")
        s = lax.axis_index("s")
        wid = s * NC + c
        zero = jnp.zeros((16,), jnp.float32)

        @pl.loop(0, N_PAD // 16)
        def _(i):
            cnt_v[pl.ds(i * 16, 16)] = zero

        pltpu.sync_copy(dst_hbm.at[pl.ds(wid * e_pt, e_pt)], dst_v)
        ones = jnp.ones((16,), jnp.float32)

        @pl.loop(0, e_pt // 16)
        def _(i):
            idx = dst_v[pl.ds(i * 16, 16)]
            plsc.addupdate_scatter(cnt_v, [idx], ones)

        pltpu.sync_copy(cnt_v, out_hbm.at[wid])

    return k(dst_p)


def _sc_aggregate(g, idx_2d, zeros):
    chunks = idx_2d.shape[1]
    stripe = N_PAD // NS
    assert chunks % NBUF == 0 and chunks >= 2 * NBUF

    @functools.partial(
        pl.kernel,
        out_type=jax.ShapeDtypeStruct((NC * N_PAD, D), jnp.float32),
        mesh=_MESH,
        scratch_types=[
            pltpu.VMEM((chunks, 2 * CHUNK), jnp.int32),
            pltpu.VMEM((NBUF, CHUNK, D), jnp.float32),
            pltpu.VMEM_SHARED((N_PAD, D), jnp.float32),
            pltpu.SemaphoreType.DMA((NBUF,)),
            pltpu.SemaphoreType.DMA((NBUF,)),
            pltpu.SemaphoreType.DMA,
        ],
    )
    def k(g_hbm, idx_hbm, z_hbm, out_hbm, idx_v, rows_v, acc, gat, scat, sem0):
        c = lax.axis_index("c")
        s = lax.axis_index("s")
        wid = s * NC + c
        r0 = s * stripe
        idx_src = idx_hbm.at[wid]
        pltpu.async_copy(idx_src, idx_v, sem0)
        pltpu.sync_copy(z_hbm.at[pl.ds(r0, stripe)], acc.at[pl.ds(r0, stripe)])
        pltpu.make_async_copy(idx_src, idx_v, sem0).wait()
        plsc.subcore_barrier()

        def issue_gather(t, b):
            pltpu.async_copy(
                g_hbm.at[idx_v.at[t, pl.ds(0, CHUNK)]], rows_v.at[b], gat.at[b])

        def wait_gather(t, b):
            pltpu.make_async_copy(
                g_hbm.at[idx_v.at[t, pl.ds(0, CHUNK)]], rows_v.at[b],
                gat.at[b]).wait()

        def issue_scatter(t, b):
            pltpu.async_copy(rows_v.at[b],
                             acc.at[idx_v.at[t, pl.ds(CHUNK, CHUNK)]],
                             scat.at[b], add=True)

        def wait_scatter(t, b):
            pltpu.make_async_copy(
                rows_v.at[b], acc.at[idx_v.at[t, pl.ds(CHUNK, CHUNK)]],
                scat.at[b]).wait()

        for t in range(_LEAD):
            issue_gather(t, t)
        for t in range(_LAG):
            wait_gather(t, t % NBUF)
            issue_scatter(t, t % NBUF)
            issue_gather(t + _LEAD, (t + _LEAD) % NBUF)

        @pl.loop(0, (chunks - NBUF) // NBUF)
        def _(i):
            for kk in range(NBUF):
                t = _LAG + i * NBUF + kk
                b = (_LAG + kk) % NBUF
                bu = kk
                wait_gather(t, b)
                issue_scatter(t, b)
                wait_scatter(t - _LAG, bu)
                issue_gather(t + _LEAD, bu)

        for tt in range(_LEAD):
            t = chunks - _LEAD + tt
            wait_gather(t, t % NBUF)
            issue_scatter(t, t % NBUF)
        for b in range(NBUF):
            wait_scatter(0, b)

        plsc.subcore_barrier()
        pltpu.sync_copy(
            acc.at[pl.ds(r0, stripe)], out_hbm.at[pl.ds(c * N_PAD + r0, stripe)]
        )

    return k(g, idx_2d, zeros)


def _dinv_from_counts(cnt):
    deg = jnp.sum(cnt, axis=0) + 1.0
    return lax.rsqrt(deg)


def _tc_g1(cnt, x_p, W1):
    def body(cnt_ref, x_ref, w_ref, o_ref):
        dinv = _dinv_from_counts(cnt_ref[...])
        h = jnp.dot(
            x_ref[...], w_ref[...],
            preferred_element_type=jnp.float32,
            precision=lax.Precision.HIGHEST,
        )
        o_ref[...] = h * dinv[:, None]

    return pl.pallas_call(
        body, out_shape=jax.ShapeDtypeStruct((N_PAD, D), jnp.float32)
    )(cnt, x_p, W1)


def _tc_mid(cnt, agg, g1, b1, W2):
    def body(cnt_ref, agg_ref, g1_ref, b1_ref, w2_ref, o_ref):
        dinv = _dinv_from_counts(cnt_ref[...])
        ssum = agg_ref[:N_PAD, :] + agg_ref[N_PAD:, :] + g1_ref[...]
        h1 = jnp.maximum(ssum * dinv[:, None] + b1_ref[...][None, :], 0.0)
        h2 = jnp.dot(
            h1, w2_ref[...],
            preferred_element_type=jnp.float32,
            precision=lax.Precision.HIGHEST,
        )
        o_ref[...] = h2 * dinv[:, None]

    return pl.pallas_call(
        body, out_shape=jax.ShapeDtypeStruct((N_PAD, D), jnp.float32)
    )(cnt, agg, g1, b1, W2)


def _tc_final(cnt, agg, g2, b2):
    def body(cnt_ref, agg_ref, g2_ref, b2_ref, o_ref):
        dinv = _dinv_from_counts(cnt_ref[...])
        ssum = agg_ref[:N_PAD, :] + agg_ref[N_PAD:, :] + g2_ref[...]
        o_ref[...] = ssum * dinv[:, None] + b2_ref[...][None, :]

    return pl.pallas_call(
        body, out_shape=jax.ShapeDtypeStruct((N_PAD, D), jnp.float32)
    )(cnt, agg, g2, b2)


def kernel(x, edge_index, W1, b1, W2, b2):
    e = edge_index.shape[1]
    e_pad = _pad_edges(e)
    chunks = e_pad // (NW * CHUNK)
    pad = jnp.full((e_pad - e,), N, dtype=jnp.int32)
    src_p = jnp.concatenate([edge_index[0], pad])
    dst_p = jnp.concatenate([edge_index[1], pad])
    idx_2d = jnp.concatenate(
        [src_p.reshape(NW, chunks, CHUNK), dst_p.reshape(NW, chunks, CHUNK)],
        axis=2,
    )
    x_p = jnp.pad(x, ((0, N_PAD - N), (0, 0)))
    zeros = jnp.zeros((N_PAD, D), jnp.float32)

    cnt = _sc_count(dst_p)
    g1 = _tc_g1(cnt, x_p, W1)
    agg1 = _sc_aggregate(g1, idx_2d, zeros)
    g2 = _tc_mid(cnt, agg1, g1, b1, W2)
    agg2 = _sc_aggregate(g2, idx_2d, zeros)
    out = _tc_final(cnt, agg2, g2, b2)
    return out[:N]
